# Initial kernel scaffold; baseline (speedup 1.0000x reference)
import jax
import jax.numpy as jnp
from jax.experimental import pallas as pl


def kernel(x, edge_index, edge_attr, meta_data, W1, b1, W2, b2, Wl1, Wr1, We1, att1, bg1, Wl2, Wr2, We2, att2, bg2, Wp1, bp1, Wp2, bp2, Wm, bm, Wf1, bf1, Wf2, bf2, Wf3, bf3):
    raise NotImplementedError("write your pallas kernel here")


# speedup vs baseline: 95.5451x; 95.5451x over previous
import functools

import jax
import jax.numpy as jnp
from jax import lax
from jax.experimental import pallas as pl
from jax.experimental.pallas import tpu as pltpu
from jax.experimental.pallas import tpu_sc as plsc

NC = 2
NS = 16
NW = NC * NS
SUB = 80
KSUB = 8
MACRO = KSUB * SUB
F = 16
NP = 51200

_mesh = functools.partial(
    plsc.VectorSubcoreMesh, core_axis_name="c", subcore_axis_name="s")
_SC_PARAMS = pltpu.CompilerParams(use_tc_tiling_on_sc=False)


def _lrelu(v, s):
  return jnp.where(v >= 0, v, s * v)


def _sc_gather(xl, xr, src3, dst3):
  NB = src3.shape[0]
  E = NB * MACRO

  @functools.partial(
      pl.kernel,
      out_type=[jax.ShapeDtypeStruct((E, F), jnp.float32),
                jax.ShapeDtypeStruct((E, F), jnp.float32)],
      mesh=_mesh(),
      scratch_types=[
          pltpu.VMEM((KSUB, SUB), jnp.int32),
          pltpu.VMEM((KSUB, SUB), jnp.int32),
          pltpu.VMEM((MACRO, F), jnp.float32),
          pltpu.VMEM((MACRO, F), jnp.float32),
          pltpu.SemaphoreType.DMA,
      ],
      compiler_params=_SC_PARAMS,
  )
  def k(xl_h, xr_h, src_h, dst_h, ol_h, or_h, sidx, didx, lrows, rrows, sem):
    wid = lax.axis_index("s") * NC + lax.axis_index("c")
    nit = NB // NW + jnp.where(wid < NB % NW, 1, 0)

    def body(j, carry):
      blk = wid + j * NW
      base = blk * MACRO
      pltpu.sync_copy(src_h.at[blk], sidx)
      pltpu.sync_copy(dst_h.at[blk], didx)
      descs = []
      for q in range(KSUB):
        descs.append(pltpu.async_copy(
            xl_h.at[sidx.at[q]], lrows.at[pl.ds(q * SUB, SUB)], sem))
        descs.append(pltpu.async_copy(
            xr_h.at[didx.at[q]], rrows.at[pl.ds(q * SUB, SUB)], sem))
      for d in descs:
        d.wait()
      pltpu.sync_copy(lrows, ol_h.at[pl.ds(base, MACRO)])
      pltpu.sync_copy(rrows, or_h.at[pl.ds(base, MACRO)])
      return carry

    lax.fori_loop(0, nit, body, 0)

  return k(xl, xr, src3, dst3)


def _sc_scatter(dst3, numer, exw):
  E = numer.shape[0]
  NB = dst3.shape[0]
  RPT = NP // NS

  @functools.partial(
      pl.kernel,
      out_type=jax.ShapeDtypeStruct((NC, 2, NP, F), jnp.float32),
      mesh=_mesh(),
      scratch_types=[
          pltpu.VMEM((KSUB, SUB), jnp.int32),
          pltpu.VMEM((MACRO, F), jnp.float32),
          pltpu.VMEM((MACRO, F), jnp.float32),
          pltpu.VMEM_SHARED((NP, F), jnp.float32),
          pltpu.VMEM_SHARED((NP, F), jnp.float32),
      ],
      compiler_params=_SC_PARAMS,
  )
  def k(dst_h, num_h, exw_h, out_h, didx, nrows, erows, accn, accd):
    cid = lax.axis_index("c")
    sid = lax.axis_index("s")
    wid = sid * NC + cid

    def zb(i, carry):
      nrows[i, :] = jnp.zeros((F,), jnp.float32)
      return carry

    lax.fori_loop(0, MACRO, zb, 0)

    def zcopy(i, carry):
      pltpu.sync_copy(nrows, accn.at[pl.ds((sid * (RPT // MACRO) + i) * MACRO,
                                           MACRO)])
      pltpu.sync_copy(nrows, accd.at[pl.ds((sid * (RPT // MACRO) + i) * MACRO,
                                           MACRO)])
      return carry

    lax.fori_loop(0, RPT // MACRO, zcopy, 0)
    plsc.subcore_barrier()
    nit = NB // NW + jnp.where(wid < NB % NW, 1, 0)

    def body(j, carry):
      blk = wid + j * NW
      base = blk * MACRO
      pltpu.sync_copy(dst_h.at[blk], didx)
      pltpu.sync_copy(num_h.at[pl.ds(base, MACRO)], nrows)
      pltpu.sync_copy(exw_h.at[pl.ds(base, MACRO)], erows)
      for q in range(KSUB):
        pltpu.sync_copy(nrows.at[pl.ds(q * SUB, SUB)],
                        accn.at[didx.at[q]], add=True)
        pltpu.sync_copy(erows.at[pl.ds(q * SUB, SUB)],
                        accd.at[didx.at[q]], add=True)
      return carry

    lax.fori_loop(0, nit, body, 0)
    plsc.subcore_barrier()
    sl = pl.ds(sid * RPT, RPT)
    pltpu.sync_copy(accn.at[sl], out_h.at[cid, 0, sl])
    pltpu.sync_copy(accd.at[sl], out_h.at[cid, 1, sl])

  return k(dst3, numer, exw)


def _dot(a, b):
  return jnp.dot(a, b, preferred_element_type=jnp.float32)


def _tc_node_prep(x8, w1, b1, w2, b2, wl, wr):
  BLK = 1024
  G = NP // BLK

  def body(x_r, w1_r, b1_r, w2_r, b2_r, wl_r, wr_r, ol_r, or_r):
    h = _lrelu(_dot(x_r[...], w1_r[...]) + b1_r[...], 0.1)
    h = _lrelu(_dot(h, w2_r[...]) + b2_r[...], 0.1)
    ol_r[...] = _dot(h, wl_r[...])
    or_r[...] = _dot(h, wr_r[...])

  full = lambda a: pl.BlockSpec(a.shape, lambda i: (0, 0))
  return pl.pallas_call(
      body,
      grid=(G,),
      in_specs=[pl.BlockSpec((BLK, 8), lambda i: (i, 0)),
                full(w1), full(b1), full(w2), full(b2), full(wl), full(wr)],
      out_specs=[pl.BlockSpec((BLK, F), lambda i: (i, 0))] * 2,
      out_shape=[jax.ShapeDtypeStruct((NP, F), jnp.float32)] * 2,
  )(x8, w1, b1, w2, b2, wl, wr)


def _tc_edge_math(xls8, xrd8, attr8, me, smat, smat_t, att_t):
  R = xls8.shape[0]
  BLK = 1000
  G = R // BLK

  def body(xl_r, xr_r, at_r, me_r, s_r, st_r, av_r, on_r, oe_r):
    ee = _dot(at_r[...], me_r[...])
    m = xl_r[...] + xr_r[...] + ee
    m = _lrelu(m, 0.2)
    t = m * av_r[...]
    l8 = _dot(t, s_r[...])
    ex8 = jnp.exp(jnp.clip(l8, -60.0, 60.0))
    exbc = _dot(ex8, st_r[...])
    on_r[...] = xl_r[...] * exbc
    oe_r[...] = exbc

  full = lambda a: pl.BlockSpec(a.shape, lambda i: (0, 0))
  return pl.pallas_call(
      body,
      grid=(G,),
      in_specs=[pl.BlockSpec((BLK, 128), lambda i: (i, 0)),
                pl.BlockSpec((BLK, 128), lambda i: (i, 0)),
                pl.BlockSpec((BLK, 16), lambda i: (i, 0)),
                full(me), full(smat), full(smat_t), full(att_t)],
      out_specs=[pl.BlockSpec((BLK, 128), lambda i: (i, 0))] * 2,
      out_shape=[jax.ShapeDtypeStruct((R, 128), jnp.float32)] * 2,
  )(xls8, xrd8, attr8, me, smat, smat_t, att_t)


def _tc_combine(n0, n1, d0, d1, bg, wl, wr):
  BLK = 1024
  G = NP // BLK

  def body(n0_r, n1_r, d0_r, d1_r, bg_r, wl_r, wr_r, ol_r, or_r):
    h = (n0_r[...] + n1_r[...]) / (d0_r[...] + d1_r[...] + 1e-16) + bg_r[...]
    h = jnp.where(h > 0, h, jnp.exp(jnp.minimum(h, 0.0)) - 1.0)
    ol_r[...] = _dot(h, wl_r[...])
    or_r[...] = _dot(h, wr_r[...])

  full = lambda a: pl.BlockSpec(a.shape, lambda i: (0, 0))
  blk = pl.BlockSpec((BLK, F), lambda i: (i, 0))
  return pl.pallas_call(
      body,
      grid=(G,),
      in_specs=[blk, blk, blk, blk, full(bg), full(wl), full(wr)],
      out_specs=[pl.BlockSpec((BLK, F), lambda i: (i, 0))] * 2,
      out_shape=[jax.ShapeDtypeStruct((NP, F), jnp.float32)] * 2,
  )(n0, n1, d0, d1, bg, wl, wr)


def _tc_node_sum(n0, n1, d0, d1, bg, n_valid):
  BLK = 1024
  G = NP // BLK

  def body(n0_r, n1_r, d0_r, d1_r, bg_r, o_r):
    i = pl.program_id(0)
    h = (n0_r[...] + n1_r[...]) / (d0_r[...] + d1_r[...] + 1e-16) + bg_r[...]
    rid = i * BLK + lax.broadcasted_iota(jnp.int32, (BLK, F), 0)
    h = jnp.where(rid < n_valid, h, 0.0)
    part = jnp.sum(h, axis=0, keepdims=True)

    @pl.when(i == 0)
    def _():
      o_r[...] = jnp.zeros((1, F), jnp.float32)

    o_r[...] += part

  full = lambda a: pl.BlockSpec(a.shape, lambda i: (0, 0))
  blk = pl.BlockSpec((BLK, F), lambda i: (i, 0))
  return pl.pallas_call(
      body,
      grid=(G,),
      in_specs=[blk, blk, blk, blk, full(bg)],
      out_specs=pl.BlockSpec((1, F), lambda i: (0, 0)),
      out_shape=jax.ShapeDtypeStruct((1, F), jnp.float32),
  )(n0, n1, d0, d1, bg)


def _tc_head(g, meta, wp1, bp1, wp2, bp2, wm, bm, wf1, bf1, wf2, bf2, wf3, bf3):
  args = (g, meta, wp1, bp1, wp2, bp2, wm, bm, wf1, bf1, wf2, bf2, wf3, bf3)

  def body(g_r, mt_r, wp1_r, bp1_r, wp2_r, bp2_r, wm_r, bm_r,
           wf1_r, bf1_r, wf2_r, bf2_r, wf3_r, bf3_r, o_r):
    gg = jnp.maximum(_dot(g_r[...], wp1_r[...]) + bp1_r[...], 0.0)
    gg = _dot(gg, wp2_r[...]) + bp2_r[...]
    mm = jnp.maximum(_dot(mt_r[...], wm_r[...]) + bm_r[...], 0.0)
    z = jnp.concatenate([gg, mm], axis=1)
    z = _lrelu(_dot(z, wf1_r[...]) + bf1_r[...], 0.1)
    z = _lrelu(_dot(z, wf2_r[...]) + bf2_r[...], 0.1)
    z = _dot(z, wf3_r[...]) + bf3_r[...]
    zmax = jnp.max(z, axis=1, keepdims=True)
    e = jnp.exp(z - zmax)
    o_r[...] = e / jnp.sum(e, axis=1, keepdims=True)

  full = lambda a: pl.BlockSpec(a.shape, lambda i: tuple(0 for _ in a.shape))
  return pl.pallas_call(
      body,
      grid=(1,),
      in_specs=[full(a) for a in args],
      out_specs=pl.BlockSpec((1, 3), lambda i: (0, 0)),
      out_shape=jax.ShapeDtypeStruct((1, 3), jnp.float32),
  )(*args)


def _edge_mats(we, att):
  me = jnp.kron(jnp.eye(8, dtype=jnp.float32), we)
  s1 = jnp.kron(jnp.eye(2, dtype=jnp.float32), jnp.ones((8, 1), jnp.float32))
  smat = jnp.kron(jnp.eye(8, dtype=jnp.float32), s1)
  att_t = jnp.tile(att.reshape(-1), 8).reshape(1, 128)
  return me, smat, smat.T, att_t


def _gat_layer(xl, xr, src3, dst3, attr8, we, att):
  xls, xrd = _sc_gather(xl, xr, src3, dst3)
  E = xls.shape[0]
  me, smat, smat_t, att_t = _edge_mats(we, att)
  numer8, exw8 = _tc_edge_math(
      xls.reshape(E // 8, 128), xrd.reshape(E // 8, 128), attr8,
      me, smat, smat_t, att_t)
  part = _sc_scatter(dst3, numer8.reshape(E, F), exw8.reshape(E, F))
  return part


def kernel(x, edge_index, edge_attr, meta_data, W1, b1, W2, b2,
           Wl1, Wr1, We1, att1, bg1, Wl2, Wr2, We2, att2, bg2,
           Wp1, bp1, Wp2, bp2, Wm, bm, Wf1, bf1, Wf2, bf2, Wf3, bf3):
  N = x.shape[0]
  E = edge_index.shape[1]

  x8 = jnp.pad(x.astype(jnp.float32), ((0, NP - N), (0, 8 - x.shape[1])))
  w1p = jnp.pad(W1, ((0, 8 - W1.shape[0]), (0, 0)))
  src3 = edge_index[0].astype(jnp.int32).reshape(E // MACRO, KSUB, SUB)
  dst3 = edge_index[1].astype(jnp.int32).reshape(E // MACRO, KSUB, SUB)
  attr8 = edge_attr.reshape(E // 8, 16)
  r2 = lambda v: v.reshape(1, -1)

  xl1, xr1 = _tc_node_prep(x8, w1p, r2(b1), W2, r2(b2), Wl1, Wr1)
  part1 = _gat_layer(xl1, xr1, src3, dst3, attr8, We1, att1)
  xl2, xr2 = _tc_combine(part1[0, 0], part1[1, 0], part1[0, 1], part1[1, 1],
                         r2(bg1), Wl2, Wr2)
  part2 = _gat_layer(xl2, xr2, src3, dst3, attr8, We2, att2)
  g = _tc_node_sum(part2[0, 0], part2[1, 0], part2[0, 1], part2[1, 1],
                   r2(bg2), N)
  return _tc_head(g, meta_data.reshape(1, -1), Wp1, r2(bp1), Wp2, r2(bp2),
                  Wm, r2(bm), Wf1, r2(bf1), Wf2, r2(bf2), Wf3, r2(bf3))


# speedup vs baseline: 101.2754x; 1.0600x over previous
import functools

import jax
import jax.numpy as jnp
import numpy as np
from jax import lax
from jax.experimental import pallas as pl
from jax.experimental.pallas import tpu as pltpu
from jax.experimental.pallas import tpu_sc as plsc

NC = 2
NS = 16
NW = NC * NS
SUB = 80
KSUB = 8
MACRO = KSUB * SUB
F = 16
NP = 51200

_mesh = functools.partial(
    plsc.VectorSubcoreMesh, core_axis_name="c", subcore_axis_name="s")
_SC_PARAMS = pltpu.CompilerParams(use_tc_tiling_on_sc=False)


def _lrelu(v, s):
  return jnp.where(v >= 0, v, s * v)


def _vperm(x, idx):
  dnums = lax.GatherDimensionNumbers(
      offset_dims=(), collapsed_slice_dims=(0,), start_index_map=(0,))
  return lax.gather(x, idx[:, None], dnums, (1,),
                    mode=lax.GatherScatterMode.PROMISE_IN_BOUNDS)


_XOR_PERMS = [np.asarray(np.arange(16) ^ d, dtype=np.int32) for d in (1, 2, 4)]


def _sc_gat_layer(xl, xr, src3, dst3, attr, cons):
  NB = src3.shape[0]
  RPT = NP // NS

  @functools.partial(
      pl.kernel,
      out_type=jax.ShapeDtypeStruct((NC, 2, NP, F), jnp.float32),
      mesh=_mesh(),
      scratch_types=[
          pltpu.VMEM((KSUB, SUB), jnp.int32),
          pltpu.VMEM((KSUB, SUB), jnp.int32),
          pltpu.VMEM((MACRO // 8, 16), jnp.float32),
          pltpu.VMEM((MACRO, F), jnp.float32),
          pltpu.VMEM((MACRO, F), jnp.float32),
          pltpu.VMEM((8, F), jnp.float32),
          pltpu.VMEM_SHARED((NP, F), jnp.float32),
          pltpu.VMEM_SHARED((NP, F), jnp.float32),
          pltpu.SemaphoreType.DMA,
      ],
      compiler_params=_SC_PARAMS,
  )
  def k(xl_h, xr_h, src_h, dst_h, attr_h, cons_h, out_h,
        sidx, didx, attv, lrows, rrows, consv, accn, accd, sem):
    cid = lax.axis_index("c")
    sid = lax.axis_index("s")
    wid = sid * NC + cid

    pltpu.sync_copy(cons_h, consv)
    we0 = consv[0, :]
    we1 = consv[1, :]
    atf = consv[2, :]
    lane = lax.iota(jnp.int32, 16)
    perms = [lane ^ d for d in (1, 2, 4)]

    def zb(i, carry):
      lrows[i, :] = jnp.zeros((F,), jnp.float32)
      return carry

    lax.fori_loop(0, MACRO, zb, 0)

    def zcopy(i, carry):
      sl = pl.ds((sid * (RPT // MACRO) + i) * MACRO, MACRO)
      pltpu.sync_copy(lrows, accn.at[sl])
      pltpu.sync_copy(lrows, accd.at[sl])
      return carry

    lax.fori_loop(0, RPT // MACRO, zcopy, 0)
    plsc.subcore_barrier()

    nit = NB // NW + jnp.where(wid < NB % NW, 1, 0)

    def body(j, carry):
      blk = wid + j * NW
      base = blk * MACRO
      pltpu.sync_copy(src_h.at[blk], sidx)
      pltpu.sync_copy(dst_h.at[blk], didx)
      pltpu.sync_copy(attr_h.at[pl.ds(base // 8, MACRO // 8)], attv)
      descs = []
      for q in range(KSUB):
        descs.append(pltpu.async_copy(
            xl_h.at[sidx.at[q]], lrows.at[pl.ds(q * SUB, SUB)], sem))
        descs.append(pltpu.async_copy(
            xr_h.at[didx.at[q]], rrows.at[pl.ds(q * SUB, SUB)], sem))
      for d in descs:
        d.wait()

      @plsc.parallel_loop(0, MACRO, unroll=2)
      def inner(e):
        arow = attv[e // 8, :]
        j2 = (e % 8) * 2
        ia0 = jnp.full((16,), j2, dtype=jnp.int32)
        a0 = _vperm(arow, ia0)
        a1 = _vperm(arow, ia0 + 1)
        lrow = lrows[e, :]
        v = lrow + rrows[e, :] + a0 * we0 + a1 * we1
        m = jnp.maximum(v, 0.2 * v)
        t = m * atf
        for p in perms:
          t = t + _vperm(t, p)
        t = jnp.clip(t, -60.0, 60.0)
        ev = jnp.exp(t)
        lrows[e, :] = lrow * ev
        rrows[e, :] = ev

      for q in range(KSUB):
        pltpu.sync_copy(lrows.at[pl.ds(q * SUB, SUB)],
                        accn.at[didx.at[q]], add=True)
        pltpu.sync_copy(rrows.at[pl.ds(q * SUB, SUB)],
                        accd.at[didx.at[q]], add=True)
      return carry

    lax.fori_loop(0, nit, body, 0)
    plsc.subcore_barrier()
    sl = pl.ds(sid * RPT, RPT)
    pltpu.sync_copy(accn.at[sl], out_h.at[cid, 0, sl])
    pltpu.sync_copy(accd.at[sl], out_h.at[cid, 1, sl])

  return k(xl, xr, src3, dst3, attr, cons)


def _dot(a, b):
  return jnp.dot(a, b, preferred_element_type=jnp.float32)


def _tc_node_prep(x8, w1, b1, w2, b2, wl, wr):
  BLK = 1024
  G = NP // BLK

  def body(x_r, w1_r, b1_r, w2_r, b2_r, wl_r, wr_r, ol_r, or_r):
    h = _lrelu(_dot(x_r[...], w1_r[...]) + b1_r[...], 0.1)
    h = _lrelu(_dot(h, w2_r[...]) + b2_r[...], 0.1)
    ol_r[...] = _dot(h, wl_r[...])
    or_r[...] = _dot(h, wr_r[...])

  full = lambda a: pl.BlockSpec(a.shape, lambda i: (0, 0))
  return pl.pallas_call(
      body,
      grid=(G,),
      in_specs=[pl.BlockSpec((BLK, 8), lambda i: (i, 0)),
                full(w1), full(b1), full(w2), full(b2), full(wl), full(wr)],
      out_specs=[pl.BlockSpec((BLK, F), lambda i: (i, 0))] * 2,
      out_shape=[jax.ShapeDtypeStruct((NP, F), jnp.float32)] * 2,
  )(x8, w1, b1, w2, b2, wl, wr)


def _tc_combine(n0, n1, d0, d1, bg, wl, wr):
  BLK = 1024
  G = NP // BLK

  def body(n0_r, n1_r, d0_r, d1_r, bg_r, wl_r, wr_r, ol_r, or_r):
    h = (n0_r[...] + n1_r[...]) / (d0_r[...] + d1_r[...] + 1e-16) + bg_r[...]
    h = jnp.where(h > 0, h, jnp.exp(jnp.minimum(h, 0.0)) - 1.0)
    ol_r[...] = _dot(h, wl_r[...])
    or_r[...] = _dot(h, wr_r[...])

  full = lambda a: pl.BlockSpec(a.shape, lambda i: (0, 0))
  blk = pl.BlockSpec((BLK, F), lambda i: (i, 0))
  return pl.pallas_call(
      body,
      grid=(G,),
      in_specs=[blk, blk, blk, blk, full(bg), full(wl), full(wr)],
      out_specs=[pl.BlockSpec((BLK, F), lambda i: (i, 0))] * 2,
      out_shape=[jax.ShapeDtypeStruct((NP, F), jnp.float32)] * 2,
  )(n0, n1, d0, d1, bg, wl, wr)


def _tc_node_sum(n0, n1, d0, d1, bg, n_valid):
  BLK = 1024
  G = NP // BLK

  def body(n0_r, n1_r, d0_r, d1_r, bg_r, o_r):
    i = pl.program_id(0)
    h = (n0_r[...] + n1_r[...]) / (d0_r[...] + d1_r[...] + 1e-16) + bg_r[...]
    rid = i * BLK + lax.broadcasted_iota(jnp.int32, (BLK, F), 0)
    h = jnp.where(rid < n_valid, h, 0.0)
    part = jnp.sum(h, axis=0, keepdims=True)

    @pl.when(i == 0)
    def _():
      o_r[...] = jnp.zeros((1, F), jnp.float32)

    o_r[...] += part

  full = lambda a: pl.BlockSpec(a.shape, lambda i: (0, 0))
  blk = pl.BlockSpec((BLK, F), lambda i: (i, 0))
  return pl.pallas_call(
      body,
      grid=(G,),
      in_specs=[blk, blk, blk, blk, full(bg)],
      out_specs=pl.BlockSpec((1, F), lambda i: (0, 0)),
      out_shape=jax.ShapeDtypeStruct((1, F), jnp.float32),
  )(n0, n1, d0, d1, bg)


def _tc_head(g, meta, wp1, bp1, wp2, bp2, wm, bm, wf1, bf1, wf2, bf2, wf3, bf3):
  args = (g, meta, wp1, bp1, wp2, bp2, wm, bm, wf1, bf1, wf2, bf2, wf3, bf3)

  def body(g_r, mt_r, wp1_r, bp1_r, wp2_r, bp2_r, wm_r, bm_r,
           wf1_r, bf1_r, wf2_r, bf2_r, wf3_r, bf3_r, o_r):
    gg = jnp.maximum(_dot(g_r[...], wp1_r[...]) + bp1_r[...], 0.0)
    gg = _dot(gg, wp2_r[...]) + bp2_r[...]
    mm = jnp.maximum(_dot(mt_r[...], wm_r[...]) + bm_r[...], 0.0)
    z = jnp.concatenate([gg, mm], axis=1)
    z = _lrelu(_dot(z, wf1_r[...]) + bf1_r[...], 0.1)
    z = _lrelu(_dot(z, wf2_r[...]) + bf2_r[...], 0.1)
    z = _dot(z, wf3_r[...]) + bf3_r[...]
    zmax = jnp.max(z, axis=1, keepdims=True)
    e = jnp.exp(z - zmax)
    o_r[...] = e / jnp.sum(e, axis=1, keepdims=True)

  full = lambda a: pl.BlockSpec(a.shape, lambda i: tuple(0 for _ in a.shape))
  return pl.pallas_call(
      body,
      grid=(1,),
      in_specs=[full(a) for a in args],
      out_specs=pl.BlockSpec((1, 3), lambda i: (0, 0)),
      out_shape=jax.ShapeDtypeStruct((1, 3), jnp.float32),
  )(*args)


def kernel(x, edge_index, edge_attr, meta_data, W1, b1, W2, b2,
           Wl1, Wr1, We1, att1, bg1, Wl2, Wr2, We2, att2, bg2,
           Wp1, bp1, Wp2, bp2, Wm, bm, Wf1, bf1, Wf2, bf2, Wf3, bf3):
  N = x.shape[0]
  E = edge_index.shape[1]

  x8 = jnp.pad(x.astype(jnp.float32), ((0, NP - N), (0, 8 - x.shape[1])))
  w1p = jnp.pad(W1, ((0, 8 - W1.shape[0]), (0, 0)))
  src3 = edge_index[0].astype(jnp.int32).reshape(E // MACRO, KSUB, SUB)
  dst3 = edge_index[1].astype(jnp.int32).reshape(E // MACRO, KSUB, SUB)
  zpad = jnp.zeros((5, F), jnp.float32)
  cons1 = jnp.concatenate([We1, att1.reshape(1, F), zpad], axis=0)
  cons2 = jnp.concatenate([We2, att2.reshape(1, F), zpad], axis=0)
  r2 = lambda v: v.reshape(1, -1)

  xl1, xr1 = _tc_node_prep(x8, w1p, r2(b1), W2, r2(b2), Wl1, Wr1)
  attr8 = edge_attr.reshape(E // 8, 16)
  part1 = _sc_gat_layer(xl1, xr1, src3, dst3, attr8, cons1)
  xl2, xr2 = _tc_combine(part1[0, 0], part1[1, 0], part1[0, 1], part1[1, 1],
                         r2(bg1), Wl2, Wr2)
  part2 = _sc_gat_layer(xl2, xr2, src3, dst3, attr8, cons2)
  g = _tc_node_sum(part2[0, 0], part2[1, 0], part2[0, 1], part2[1, 1],
                   r2(bg2), N)
  return _tc_head(g, meta_data.reshape(1, -1), Wp1, r2(bp1), Wp2, r2(bp2),
                  Wm, r2(bm), Wf1, r2(bf1), Wf2, r2(bf2), Wf3, r2(bf3))
